# baseline (speedup 1.0000x reference)
import functools
import jax
import jax.numpy as jnp
from jax import lax
from jax.experimental import pallas as pl
from jax.experimental.pallas import tpu as pltpu, tpu_sc as plsc

N = 4096
G = 4
D = 256
THR = -1.0
BM = 1024

_mesh = plsc.VectorSubcoreMesh(core_axis_name="c", subcore_axis_name="s")


@functools.partial(
    pl.kernel,
    mesh=_mesh,
    out_type=jax.ShapeDtypeStruct((G, 16), jnp.int32),
    scratch_types=[
        pltpu.VMEM((N,), jnp.int32),
        pltpu.VMEM((G, 16), jnp.int32),
    ],
)
def _ends_sc(seg_hbm, out_hbm, seg_v, acc_v):
    c = lax.axis_index("c")
    s = lax.axis_index("s")

    @pl.when((c == 0) & (s == 0))
    def _():
        pltpu.sync_copy(seg_hbm, seg_v)
        for g in range(G):
            acc_v[g, :] = jnp.zeros((16,), jnp.int32)

        def body(k, carry):
            sg = seg_v[pl.ds(k * 16, 16)]
            for g in range(G):
                acc_v[g, :] += jnp.where(sg <= g, 1, 0).astype(jnp.int32)
            return carry

        lax.fori_loop(0, N // 16, body, 0)
        pltpu.sync_copy(acc_v, out_hbm)


def _edge_kernel(
    x_ref, w_ref, b_ref, seg_ref, ends_ref, score_ref, adj_ref, bne_ref, h_scr, sq_scr
):
    t = pl.program_id(0)

    @pl.when(t == 0)
    def _():
        x = x_ref[...]
        h = jnp.dot(x, w_ref[...], preferred_element_type=jnp.float32) + b_ref[...]
        h_scr[...] = h
        ones = jnp.ones((1, D), jnp.float32)
        sq_scr[...] = jax.lax.dot_general(
            ones, h * h, (((1,), (1,)), ((), ())), preferred_element_type=jnp.float32
        )
        bne_ref[...] = jnp.zeros((1, 1, 128), jnp.int32)

    @pl.when(t > 0)
    def _():
        i = t - 1
        hi = h_scr[pl.ds(i * BM, BM), :]
        hfull = h_scr[...]
        dot = jax.lax.dot_general(
            hi + hi, hfull, (((1,), (1,)), ((), ())), preferred_element_type=jnp.float32
        )
        sqi = jnp.sum(hi * hi, axis=1, keepdims=True)
        score = dot - sqi - sq_scr[...]
        score_ref[...] = score

        ends = ends_ref[...][0:1, :G]
        row = i * BM + jax.lax.broadcasted_iota(jnp.int32, (BM, 1), 0)
        col = jax.lax.broadcasted_iota(jnp.int32, (1, N), 1)
        segr = jnp.sum((row >= ends).astype(jnp.int32), axis=1, keepdims=True)
        segc = seg_ref[...]
        adj = (score > THR) & (segr == segc) & (row != col)
        adj_ref[...] = adj.astype(jnp.int8)

        rowdeg = jnp.sum(adj.astype(jnp.int32), axis=1, keepdims=True)
        lanes = jax.lax.broadcasted_iota(jnp.int32, (1, 128), 1)
        contrib = jnp.sum(jnp.where(segr == lanes, rowdeg, 0), axis=0, keepdims=True)
        bne_ref[...] += contrib.reshape(1, 1, 128)


def kernel(t, dynamicVariable, segment_ids, W, b):
    x = dynamicVariable
    b2 = b.reshape(1, D)
    seg1d = segment_ids.astype(jnp.int32)
    seg2d = seg1d.reshape(1, N)

    ends16 = _ends_sc(seg1d)
    ends128 = jnp.zeros((1, 128), jnp.int32).at[0, :G].set(jnp.sum(ends16, axis=1))

    nb = N // BM
    score, adj, bne3 = pl.pallas_call(
        _edge_kernel,
        grid=(nb + 1,),
        in_specs=[
            pl.BlockSpec((N, D), lambda t: (0, 0)),
            pl.BlockSpec((D, D), lambda t: (0, 0)),
            pl.BlockSpec((1, D), lambda t: (0, 0)),
            pl.BlockSpec((1, N), lambda t: (0, 0)),
            pl.BlockSpec((1, 128), lambda t: (0, 0)),
        ],
        out_specs=[
            pl.BlockSpec((BM, N), lambda t: (jnp.maximum(t - 1, 0), 0)),
            pl.BlockSpec((BM, N), lambda t: (jnp.maximum(t - 1, 0), 0)),
            pl.BlockSpec((1, 1, 128), lambda t: (0, 0, 0)),
        ],
        out_shape=[
            jax.ShapeDtypeStruct((N, N), jnp.float32),
            jax.ShapeDtypeStruct((N, N), jnp.int8),
            jax.ShapeDtypeStruct((1, 1, 128), jnp.int32),
        ],
        scratch_shapes=[
            pltpu.VMEM((N, D), jnp.float32),
            pltpu.VMEM((1, N), jnp.float32),
        ],
    )(x, W, b2, seg2d, ends128)

    bne = bne3.reshape(128)[:G]
    return (score, adj.astype(jnp.bool_), bne)


# speedup vs baseline: 1.4161x; 1.4161x over previous
import jax
import jax.numpy as jnp
from jax.experimental import pallas as pl
from jax.experimental.pallas import tpu as pltpu

N = 4096
G = 4
D = 256
THR = -1.0
BM = 1024


def _edge_kernel(
    x_ref, w_ref, b_ref, seg_ref, score_ref, adj_ref, bne_ref, h_scr, sq_scr, ends_scr
):
    t = pl.program_id(0)

    @pl.when(t == 0)
    def _():
        x = x_ref[...]
        h = jnp.dot(x, w_ref[...], preferred_element_type=jnp.float32) + b_ref[...]
        h_scr[...] = h
        ones = jnp.ones((1, D), jnp.float32)
        sq_scr[...] = jax.lax.dot_general(
            ones, h * h, (((1,), (1,)), ((), ())), preferred_element_type=jnp.float32
        )
        seg_full = seg_ref[...]
        e = jnp.int32(0)
        for k in range(G):
            e = e + jnp.sum((seg_full == k).astype(jnp.int32))
            ends_scr[k] = e
        bne_ref[...] = jnp.zeros((1, 1, 128), jnp.int32)

    @pl.when(t > 0)
    def _():
        i = t - 1
        hi = h_scr[pl.ds(i * BM, BM), :]
        hfull = h_scr[...]
        dot = jax.lax.dot_general(
            hi + hi, hfull, (((1,), (1,)), ((), ())), preferred_element_type=jnp.float32
        )
        sqi = jnp.sum(hi * hi, axis=1, keepdims=True)
        score = dot - sqi - sq_scr[...]
        score_ref[...] = score

        ends = [ends_scr[k] for k in range(G)]
        row = i * BM + jax.lax.broadcasted_iota(jnp.int32, (BM, 1), 0)
        col = jax.lax.broadcasted_iota(jnp.int32, (1, N), 1)
        segr = sum((row >= ends[k]).astype(jnp.int32) for k in range(G))
        segc = seg_ref[...]
        adj = (score > THR) & (segr == segc) & (row != col)
        adj_ref[...] = adj.astype(jnp.int8)

        rowdeg = jnp.sum(adj.astype(jnp.int32), axis=1, keepdims=True)
        lanes = jax.lax.broadcasted_iota(jnp.int32, (1, 128), 1)
        contrib = jnp.sum(jnp.where(segr == lanes, rowdeg, 0), axis=0, keepdims=True)
        bne_ref[...] += contrib.reshape(1, 1, 128)


def kernel(t, dynamicVariable, segment_ids, W, b):
    x = dynamicVariable
    b2 = b.reshape(1, D)
    seg2d = segment_ids.reshape(1, N).astype(jnp.int32)

    nb = N // BM
    score, adj, bne3 = pl.pallas_call(
        _edge_kernel,
        grid=(nb + 1,),
        in_specs=[
            pl.BlockSpec((N, D), lambda t: (0, 0)),
            pl.BlockSpec((D, D), lambda t: (0, 0)),
            pl.BlockSpec((1, D), lambda t: (0, 0)),
            pl.BlockSpec((1, N), lambda t: (0, 0)),
        ],
        out_specs=[
            pl.BlockSpec((BM, N), lambda t: (jnp.maximum(t - 1, 0), 0)),
            pl.BlockSpec((BM, N), lambda t: (jnp.maximum(t - 1, 0), 0)),
            pl.BlockSpec((1, 1, 128), lambda t: (0, 0, 0)),
        ],
        out_shape=[
            jax.ShapeDtypeStruct((N, N), jnp.float32),
            jax.ShapeDtypeStruct((N, N), jnp.int8),
            jax.ShapeDtypeStruct((1, 1, 128), jnp.int32),
        ],
        scratch_shapes=[
            pltpu.VMEM((N, D), jnp.float32),
            pltpu.VMEM((1, N), jnp.float32),
            pltpu.SMEM((G,), jnp.int32),
        ],
    )(x, W, b2, seg2d)

    bne = bne3.reshape(128)[:G]
    return (score, adj.astype(jnp.bool_), bne)
